# Initial kernel scaffold; baseline (speedup 1.0000x reference)
import jax
import jax.numpy as jnp
from jax.experimental import pallas as pl


def kernel(x, edge_index, W1_self, W1_nbr, g1, b1, W2_self, W2_nbr, g2, b2, Wf, gf, bf):
    raise NotImplementedError("write your pallas kernel here")


# speedup vs baseline: 4.6643x; 4.6643x over previous
import functools

import jax
import jax.numpy as jnp
from jax import lax
from jax.experimental import pallas as pl
from jax.experimental.pallas import tpu as pltpu
from jax.experimental.pallas import tpu_sc as plsc

N = 10000
C = 128
E = 320000
EPS = 1e-5

NC = 2
NS = 16
NW = NC * NS
CH = 128
NCH = 79
EPW = NCH * CH
EPAD = EPW * NW
ACC_ROWS = 10240
ZR = ACC_ROWS // NS


def _prep_edges(edge_index):
    src = edge_index[0]
    dst = edge_index[1]
    pad = EPAD - E
    src_p = jnp.concatenate([src, jnp.zeros((pad,), src.dtype)])
    dst_p = jnp.concatenate([dst, jnp.full((pad,), N, dst.dtype)])
    return src_p.reshape(NW, NCH, CH), dst_p.reshape(NW, NCH, CH)


def _seg_sum_sc(x, src3, dst3, zeros):
    mesh = plsc.VectorSubcoreMesh(core_axis_name="c", subcore_axis_name="s")

    @functools.partial(
        pl.kernel,
        out_type=jax.ShapeDtypeStruct((NC, ACC_ROWS, C), jnp.float32),
        mesh=mesh,
        scratch_types=[
            pltpu.VMEM((NCH, CH), jnp.int32),
            pltpu.VMEM((NCH, CH), jnp.int32),
            pltpu.VMEM((CH, C), jnp.float32),
            pltpu.VMEM_SHARED((ACC_ROWS, C), jnp.float32),
            pltpu.SemaphoreType.DMA,
        ],
    )
    def seg_sum(x_hbm, src_hbm, dst_hbm, z_hbm, out_hbm,
                src_v, dst_v, rows_v, acc, sem):
        cid = lax.axis_index("c")
        sid = lax.axis_index("s")
        w = cid * NS + sid
        tb = sid * ZR
        pltpu.sync_copy(z_hbm, acc.at[pl.ds(tb, ZR)])
        pltpu.sync_copy(src_hbm.at[w], src_v)
        pltpu.sync_copy(dst_hbm.at[w], dst_v)
        plsc.subcore_barrier()

        def step(j, carry):
            pltpu.async_copy(x_hbm.at[src_v.at[j]], rows_v, sem).wait()
            pltpu.sync_copy(rows_v, acc.at[dst_v.at[j]], add=True)
            return carry

        lax.fori_loop(0, NCH, step, 0, unroll=False)
        plsc.subcore_barrier()
        pltpu.sync_copy(acc.at[pl.ds(tb, ZR)], out_hbm.at[cid, pl.ds(tb, ZR)])

    return seg_sum(x, src3, dst3, zeros)


def _bn(h, g, b):
    m = jnp.mean(h, axis=0, keepdims=True)
    v = jnp.mean((h - m) ** 2, axis=0, keepdims=True)
    return (h - m) * lax.rsqrt(v + EPS) * g + b


def _tc1(x, parts, W1s, W1n, g1, b1):
    def body(x_ref, p_ref, ws_ref, wn_ref, g_ref, b_ref, o_ref):
        a = p_ref[0, :N, :] + p_ref[1, :N, :]
        h = jnp.dot(x_ref[...], ws_ref[...], preferred_element_type=jnp.float32, precision=lax.Precision.HIGHEST)
        h = h + jnp.dot(a, wn_ref[...], preferred_element_type=jnp.float32, precision=lax.Precision.HIGHEST)
        o_ref[...] = jnp.maximum(_bn(h, g_ref[...], b_ref[...]), 0.0)

    return pl.pallas_call(
        body, out_shape=jax.ShapeDtypeStruct((N, C), jnp.float32),
    )(x, parts, W1s, W1n, g1.reshape(1, C), b1.reshape(1, C))


def _tc2(local, parts, W2s, W2n, g2, b2, Wf, gf, bf):
    def body(l_ref, p_ref, ws_ref, wn_ref, g2_ref, b2_ref,
             wf_ref, gf_ref, bf_ref, o_ref):
        lcl = l_ref[...]
        a = p_ref[0, :N, :] + p_ref[1, :N, :]
        h = jnp.dot(lcl, ws_ref[...], preferred_element_type=jnp.float32, precision=lax.Precision.HIGHEST)
        h = h + jnp.dot(a, wn_ref[...], preferred_element_type=jnp.float32, precision=lax.Precision.HIGHEST)
        edge = _bn(h, g2_ref[...], b2_ref[...])
        eh = jnp.abs(lcl - edge)
        f = jnp.dot(lcl, wf_ref[:C, :], preferred_element_type=jnp.float32, precision=lax.Precision.HIGHEST)
        f = f + jnp.dot(eh, wf_ref[C:, :], preferred_element_type=jnp.float32, precision=lax.Precision.HIGHEST)
        o_ref[...] = jnp.maximum(_bn(f, gf_ref[...], bf_ref[...]), 0.0)

    return pl.pallas_call(
        body, out_shape=jax.ShapeDtypeStruct((N, C), jnp.float32),
    )(local, parts, W2s, W2n, g2.reshape(1, C), b2.reshape(1, C),
      Wf, gf.reshape(1, C), bf.reshape(1, C))


def kernel(x, edge_index, W1_self, W1_nbr, g1, b1,
           W2_self, W2_nbr, g2, b2, Wf, gf, bf):
    src3, dst3 = _prep_edges(edge_index)
    zeros = jnp.zeros((ZR, C), jnp.float32)
    p1 = _seg_sum_sc(x, src3, dst3, zeros)
    local = _tc1(x, p1, W1_self, W1_nbr, g1, b1)
    p2 = _seg_sum_sc(local, src3, dst3, zeros)
    return _tc2(local, p2, W2_self, W2_nbr, g2, b2, Wf, gf, bf)
